# baseline (speedup 1.0000x reference)
import functools

import jax
import jax.numpy as jnp
import numpy as np
import math
from jax import lax
from jax.experimental import pallas as pl
from jax.experimental.pallas import tpu as pltpu
from jax.experimental.pallas import tpu_sc as plsc

N_ = 10000
E_ = 160000
B_ = 16
NF_, PH_, OH_, CF_ = 3, 16, 2, 8
H_, DE_, NL_, CNL_ = 128, 32, 5, 3
EMB_, MAXE_, T_ = 16, 30, 200
F_ = H_ + DE_
EE_ = E_ + N_
D_ = 144

NTILES = 32
CH_G = 112
KCH_G = 48
EPAD = NTILES * CH_G * KCH_G
CH_S = 96
KCH_S = EPAD // 2 // 16 // CH_S
RP_ = N_ // 16

NBLK = 2000
EBLK = 512


def _mesh():
    return plsc.VectorSubcoreMesh(core_axis_name="c", subcore_axis_name="s")


def _sc_gather_fused(ta, tb, row, col):
    NV = D_ // 16

    @functools.partial(
        pl.kernel,
        mesh=_mesh(),
        compiler_params=pltpu.CompilerParams(use_tc_tiling_on_sc=False),
        out_type=jax.ShapeDtypeStruct((EPAD, D_), jnp.float32),
        scratch_types=(
            [pltpu.VMEM((CH_G,), jnp.int32)] * 8
            + [pltpu.VMEM((CH_G, D_), jnp.float32)] * 6
            + [pltpu.SemaphoreType.DMA] * 14
        ),
    )
    def k(ta_h, tb_h, row_h, col_h, g_h, *scr):
        ira = scr[0:4]
        irb = scr[4:8]
        bufa = scr[8:12]
        bufb = scr[12:14]
        isem = scr[14:18]
        ga = scr[18:22]
        gb = scr[22:24]
        wsem = scr[24:28]
        wid = lax.axis_index("s") * 2 + lax.axis_index("c")
        base_w = wid * (CH_G * KCH_G)

        def cbase(c):
            return base_w + c * CH_G

        def vadd(ba, bb):
            def vrow(r, carry):
                for cc in range(NV):
                    sl = pl.ds(cc * 16, 16)
                    ba[r, sl] = ba[r, sl] + bb[r, sl]
                return carry
            lax.fori_loop(0, CH_G, vrow, 0, unroll=2)

        def start_idx(c, u):
            pltpu.async_copy(row_h.at[pl.ds(cbase(c), CH_G)], ira[u], isem[u])
            pltpu.async_copy(col_h.at[pl.ds(cbase(c), CH_G)], irb[u], isem[u])

        def wait_idx(c, u):
            pltpu.make_async_copy(row_h.at[pl.ds(cbase(c), CH_G)], ira[u], isem[u]).wait()
            pltpu.make_async_copy(col_h.at[pl.ds(cbase(c), CH_G)], irb[u], isem[u]).wait()

        def start_gather(u2, u):
            pltpu.async_copy(ta_h.at[ira[u]], bufa[u], ga[u])
            pltpu.async_copy(tb_h.at[irb[u]], bufb[u2], gb[u2])

        def wait_gather(u2, u):
            pltpu.make_async_copy(ta_h.at[ira[u]], bufa[u], ga[u]).wait()
            pltpu.make_async_copy(tb_h.at[irb[u]], bufb[u2], gb[u2]).wait()

        def start_write(c, u):
            pltpu.async_copy(bufa[u], g_h.at[pl.ds(cbase(c), CH_G)], wsem[u])

        def wait_write(u):
            pltpu.make_async_copy(bufa[u], g_h.at[pl.ds(0, CH_G)], wsem[u]).wait()

        for u in range(2):
            pltpu.sync_copy(row_h.at[pl.ds(cbase(u), CH_G)], ira[u])
            pltpu.sync_copy(col_h.at[pl.ds(cbase(u), CH_G)], irb[u])
        start_idx(2, 2)
        start_idx(3, 3)
        start_gather(0, 0)
        start_gather(1, 1)
        scrib = EE_ + wid * (2 * CH_G)
        pltpu.async_copy(bufa[2], g_h.at[pl.ds(scrib, CH_G)], wsem[2])
        pltpu.async_copy(bufa[3], g_h.at[pl.ds(scrib + CH_G, CH_G)], wsem[3])

        def ops(c, u, do_bg):
            wait_gather(u % 2, u)
            if do_bg:
                start_idx(c + 4, u)
            vadd(bufa[u], bufb[u % 2])
            start_write(c, u)
            if do_bg:
                wait_idx(c + 2, (u + 2) % 4)
                wait_write((u + 2) % 4)
                start_gather(u % 2, (u + 2) % 4)

        def body(p, carry):
            for u in range(4):
                ops(p * 4 + u, u, True)
            return carry

        lax.fori_loop(0, (KCH_G - 4) // 4, body, 0)

        for u in range(4):
            c = KCH_G - 4 + u
            wait_gather(u % 2, u)
            vadd(bufa[u], bufb[u % 2])
            if u < 2:
                wait_idx(c + 2, (u + 2) % 4)
                wait_write((u + 2) % 4)
                start_gather(u % 2, (u + 2) % 4)
            start_write(c, u)
        for u in range(4):
            wait_write(u)

    return k(ta, tb, row, col)


def _sc_scatter(vals, row, zer):
    @functools.partial(
        pl.kernel,
        mesh=_mesh(),
        compiler_params=pltpu.CompilerParams(use_tc_tiling_on_sc=False),
        out_type=jax.ShapeDtypeStruct((2, N_, D_), jnp.float32),
        scratch_types=[
            pltpu.VMEM((CH_S,), jnp.int32),
            pltpu.VMEM((CH_S,), jnp.int32),
            pltpu.VMEM((2 * CH_S, D_), jnp.float32),
            pltpu.VMEM_SHARED((N_, D_), jnp.float32),
            pltpu.SemaphoreType.DMA,
            pltpu.SemaphoreType.DMA,
        ],
    )
    def k(vals_h, row_h, zer_h, out_h, idx0, idx1, vbuf, acc, ls0, ls1):
        cid = lax.axis_index("c")
        sid = lax.axis_index("s")
        pltpu.sync_copy(zer_h.at[pl.ds(sid * RP_, RP_)], acc.at[pl.ds(sid * RP_, RP_)])
        plsc.subcore_barrier()
        base_c = cid * (EPAD // 2) + sid * (CH_S * KCH_S)
        idx = (idx0, idx1)
        sem = (ls0, ls1)

        def cbase(c):
            return base_c + c * CH_S

        def start_loads(c, s):
            pltpu.async_copy(row_h.at[pl.ds(cbase(c), CH_S)], idx[s], sem[s])
            pltpu.async_copy(vals_h.at[pl.ds(cbase(c), CH_S)],
                             vbuf.at[pl.ds(s * CH_S, CH_S)], sem[s])

        def wait_loads(s):
            pltpu.make_async_copy(row_h.at[pl.ds(0, CH_S)], idx[s], sem[s]).wait()
            pltpu.make_async_copy(vals_h.at[pl.ds(0, CH_S)],
                                  vbuf.at[pl.ds(s * CH_S, CH_S)], sem[s]).wait()

        def scat(s):
            pltpu.sync_copy(vbuf.at[pl.ds(s * CH_S, CH_S)], acc.at[idx[s]], add=True)

        start_loads(0, 0)

        def body(p, carry):
            start_loads(2 * p + 1, 1)
            wait_loads(0)
            scat(0)
            start_loads(2 * p + 2, 0)
            wait_loads(1)
            scat(1)
            return carry

        lax.fori_loop(0, KCH_S // 2 - 1, body, 0)
        start_loads(KCH_S - 1, 1)
        wait_loads(0)
        scat(0)
        wait_loads(1)
        scat(1)
        plsc.subcore_barrier()

        def wb(j, carry):
            rb = pl.multiple_of(sid * RP_ + j * 125, 125)
            pltpu.sync_copy(acc.at[pl.ds(rb, 125)], vbuf.at[pl.ds(0, 125)])
            pltpu.sync_copy(vbuf.at[pl.ds(0, 125)], out_h.at[cid, pl.ds(rb, 125)])
            return carry

        lax.fori_loop(0, RP_ // 125, wb, 0)

    return k(vals, row, zer)


def _full(shape):
    return pl.BlockSpec(shape, lambda *_: tuple(0 for _ in shape))


def _rows(blk, width):
    return pl.BlockSpec((blk, width), lambda i: (i, 0))


def _silu(x):
    return x * jax.nn.sigmoid(x)


def _k0_time_embed(ts2, pe, d0w, d0b, d1w, d1b):
    def body(ts_r, pe_r, d0w_r, d0b_r, d1w_r, d1b_r, o_r):
        oh = (ts_r[...] == lax.broadcasted_iota(jnp.int32, (B_, T_), 1))
        pet = jnp.where(oh, 1.0, 0.0) @ pe_r[...]

        def mish(v):
            return v * jnp.tanh(jax.nn.softplus(v))

        z = mish(pet @ d0w_r[...] + d0b_r[...])
        o_r[...] = z @ d1w_r[...] + d1b_r[...]

    return pl.pallas_call(
        body,
        out_shape=jax.ShapeDtypeStruct((B_, DE_), jnp.float32),
        in_specs=[_full((B_, 1)), _full((T_, DE_)), _full((DE_, DE_ * 4)),
                  _full((1, DE_ * 4)), _full((DE_ * 4, DE_)), _full((1, DE_))],
        out_specs=_full((B_, DE_)),
    )(ts2, pe, d0w, d0b, d1w, d1b)


def _k1_prelude(xf, cf, ids2, batch2, te_b, wn, bn, wc1, wc2f, bc):
    def body(xf_r, cf_r, ids_r, b_r, te_r, wn_r, bn_r, wc1_r, wc2f_r, bc_r, hv_r, hc_r):
        a = xf_r[...] @ wn_r[...] + bn_r[...]
        ohb = jnp.where(b_r[...] == lax.broadcasted_iota(jnp.int32, (NBLK, B_), 1), 1.0, 0.0)
        te = ohb @ te_r[...]
        hv_r[...] = jnp.concatenate([a, te], axis=1)
        ohi = jnp.where(ids_r[...] == lax.broadcasted_iota(jnp.int32, (NBLK, MAXE_), 1), 1.0, 0.0)
        hc_r[...] = cf_r[...] @ wc1_r[...] + ohi @ wc2f_r[...] + bc_r[...]

    return pl.pallas_call(
        body,
        grid=(N_ // NBLK,),
        out_shape=[jax.ShapeDtypeStruct((N_, F_), jnp.float32),
                   jax.ShapeDtypeStruct((N_, H_), jnp.float32)],
        in_specs=[_rows(NBLK, NF_ * PH_), _rows(NBLK, OH_ * CF_), _rows(NBLK, 1),
                  _rows(NBLK, 1), _full((B_, DE_)), _full((NF_ * PH_, H_)),
                  _full((1, H_)), _full((OH_ * CF_, H_)), _full((MAXE_, H_)),
                  _full((1, H_))],
        out_specs=[_rows(NBLK, F_), _rows(NBLK, H_)],
    )(xf, cf, ids2, batch2, te_b, wn, bn, wc1, wc2f, bc)


def _k2_node_pre(h, coord, wa, wb, film):
    fin = h.shape[1]

    if film is None:
        def body(h_r, c_r, wa_r, wb_r, ta_r, tb_r):
            hp = h_r[...]
            cpad = jnp.pad(c_r[...], ((0, 0), (0, D_ - H_ - 3)))
            ta_r[...] = jnp.concatenate([hp @ wa_r[...], cpad], axis=1)
            tb_r[...] = jnp.concatenate([hp @ wb_r[...], -cpad], axis=1)

        return pl.pallas_call(
            body,
            grid=(N_ // NBLK,),
            out_shape=[jax.ShapeDtypeStruct((N_, D_), jnp.float32),
                       jax.ShapeDtypeStruct((N_, D_), jnp.float32)],
            in_specs=[_rows(NBLK, fin), _rows(NBLK, 3), _full((fin, H_)), _full((fin, H_))],
            out_specs=[_rows(NBLK, D_), _rows(NBLK, D_)],
        )(h, coord, wa, wb), h

    sc_l, bi_l, batch2 = film

    def body(h_r, c_r, b_r, sc_r, bi_r, wa_r, wb_r, ta_r, tb_r, hp_r):
        ohb = jnp.where(b_r[...] == lax.broadcasted_iota(jnp.int32, (NBLK, B_), 1), 1.0, 0.0)
        hp = (ohb @ sc_r[...]) * h_r[...] + ohb @ bi_r[...]
        hp_r[...] = hp
        cpad = jnp.pad(c_r[...], ((0, 0), (0, D_ - H_ - 3)))
        ta_r[...] = jnp.concatenate([hp @ wa_r[...], cpad], axis=1)
        tb_r[...] = jnp.concatenate([hp @ wb_r[...], -cpad], axis=1)

    ta, tb, hp = pl.pallas_call(
        body,
        grid=(N_ // NBLK,),
        out_shape=[jax.ShapeDtypeStruct((N_, D_), jnp.float32),
                   jax.ShapeDtypeStruct((N_, D_), jnp.float32),
                   jax.ShapeDtypeStruct((N_, fin), jnp.float32)],
        in_specs=[_rows(NBLK, fin), _rows(NBLK, 3), _rows(NBLK, 1),
                  _full((B_, fin)), _full((B_, fin)), _full((fin, H_)), _full((fin, H_))],
        out_specs=[_rows(NBLK, D_), _rows(NBLK, D_), _rows(NBLK, fin)],
    )(h, coord, batch2, sc_l, bi_l, wa, wb)
    return (ta, tb), hp


def _k3_edge(gg, ea2, aux, w1, wc0, c2, normalize):
    def body(g_r, ea_r, aux_r, w1_r, wc0_r, c2_r, m_r):
        pid = pl.program_id(0)
        g = g_r[...]
        cd = g[:, H_:H_ + 3]
        radial = jnp.sum(cd * cd, axis=1, keepdims=True)
        aux = aux_r[...]
        g0 = (g[:, :H_] + radial * aux[0:1, :] + ea_r[...] * aux[1:2, :]
              + aux[2:3, :])
        m0 = _silu(g0)
        m = _silu(m0 @ w1_r[...] + aux[3:4, :])
        q = _silu(m @ wc0_r[...] + aux[4:5, :])
        t = q @ c2_r[...]
        if normalize:
            cdn = cd / (jnp.sqrt(radial) + 1e-8)
        else:
            cdn = cd
        trans = cdn * t
        gi = pid * EBLK + lax.broadcasted_iota(jnp.int32, (EBLK, 1), 0)
        valid = jnp.where(gi < EE_, 1.0, 0.0)
        tail = jnp.pad(jnp.concatenate([trans, valid], axis=1),
                       ((0, 0), (0, D_ - H_ - 4)))
        full = jnp.concatenate([m, tail], axis=1)
        m_r[...] = jnp.where(gi < EE_, full, 0.0)

    return pl.pallas_call(
        body,
        grid=(EPAD // EBLK,),
        out_shape=jax.ShapeDtypeStruct((EPAD, D_), jnp.float32),
        in_specs=[_rows(EBLK, D_), _rows(EBLK, 1),
                  _full((8, H_)), _full((H_, H_)), _full((H_, H_)), _full((H_, 1))],
        out_specs=_rows(EBLK, D_),
    )(gg, ea2, aux, w1, wc0, c2)


def _k4_node_post(h, coord, p0, p1, w0a, w0b, b0, w1n, b1n):
    fin = h.shape[1]

    def body(h_r, c_r, p0_r, p1_r, w0a_r, w0b_r, b0_r, w1_r, b1_r, ho_r, co_r):
        agg = p0_r[...] + p1_r[...]
        cnt = jnp.maximum(agg[:, H_ + 3:H_ + 4], 1.0)
        co_r[...] = c_r[...] + agg[:, H_:H_ + 3] / cnt
        z = _silu(h_r[...] @ w0a_r[...] + agg[:, :H_] @ w0b_r[...] + b0_r[...])
        ho_r[...] = h_r[...] + z @ w1_r[...] + b1_r[...]

    return pl.pallas_call(
        body,
        grid=(N_ // NBLK,),
        out_shape=[jax.ShapeDtypeStruct((N_, fin), jnp.float32),
                   jax.ShapeDtypeStruct((N_, 3), jnp.float32)],
        in_specs=[_rows(NBLK, fin), _rows(NBLK, 3), _rows(NBLK, D_), _rows(NBLK, D_),
                  _full((fin, H_)), _full((H_, H_)), _full((1, H_)),
                  _full((H_, fin)), _full((1, fin))],
        out_specs=[_rows(NBLK, fin), _rows(NBLK, 3)],
    )(h, coord, p0, p1, w0a, w0b, b0, w1n, b1n)


def _k5a_pool(hc, batch2, wco, bco):
    def body(hc_r, b_r, wco_r, bco_r, g_r):
        i = pl.program_id(0)
        hco = hc_r[...] @ wco_r[...] + bco_r[...]
        bv = b_r[...]
        parts = []
        for b in range(B_):
            mb = jnp.where(bv == b, hco, -jnp.inf)
            parts.append(jnp.max(mb, axis=0, keepdims=True))
        cur = jnp.concatenate(parts, axis=0)

        @pl.when(i == 0)
        def _():
            g_r[...] = jnp.full((B_, H_), -jnp.inf, jnp.float32)

        g_r[...] = jnp.maximum(g_r[...], cur)

    return pl.pallas_call(
        body,
        grid=(N_ // NBLK,),
        out_shape=jax.ShapeDtypeStruct((B_, H_), jnp.float32),
        in_specs=[_rows(NBLK, H_), _rows(NBLK, 1), _full((H_, H_)), _full((1, H_))],
        out_specs=_full((B_, H_)),
    )(hc, batch2, wco, bco)


def _k5b_film_tables(g, wfc, bfc):
    cc = 2 * NL_ * F_

    def body(g_r, w_r, b_r, o_r):
        gv = g_r[...]
        gv = jnp.where(gv > -1e30, gv, 0.0)
        o_r[...] = gv @ w_r[...] + b_r[...]

    return pl.pallas_call(
        body,
        out_shape=jax.ShapeDtypeStruct((B_, cc), jnp.float32),
        in_specs=[_full((B_, H_)), _full((H_, cc)), _full((1, cc))],
        out_specs=_full((B_, cc)),
    )(g, wfc, bfc)


def _k6_out(h, w0, b0, w1, b1, w2, b2):
    def body(h_r, w0_r, b0_r, w1_r, b1_r, w2_r, b2_r, o_r):
        z = jnp.maximum(h_r[...] @ w0_r[...] + b0_r[...], 0.0)
        z = jnp.maximum(z @ w1_r[...] + b1_r[...], 0.0)
        o_r[...] = z @ w2_r[...] + b2_r[...]

    return pl.pallas_call(
        body,
        grid=(N_ // NBLK,),
        out_shape=jax.ShapeDtypeStruct((N_, NF_ * PH_), jnp.float32),
        in_specs=[_rows(NBLK, F_), _full((F_, H_)), _full((1, H_)),
                  _full((H_, H_)), _full((1, H_)), _full((H_, NF_ * PH_)),
                  _full((1, NF_ * PH_))],
        out_specs=_rows(NBLK, NF_ * PH_),
    )(h, w0, b0, w1, b1, w2, b2)


def _pe_table():
    pes = np.zeros((T_, DE_), dtype=np.float32)
    pos = np.arange(T_, dtype=np.float32)[:, None] + 1.0
    div = np.exp(np.arange(0, DE_, 2, dtype=np.float32) * (-math.log(10000.0) / DE_))
    pes[:, 0::2] = np.sin(pos * div)
    pes[:, 1::2] = np.cos(pos * div)
    return jnp.asarray(pes)


def _egcl_prep(p, fin, ed, eemb_w, eemb_b):
    w0 = p["e0"]["w"]
    wa = w0[:, :fin].T
    wb = w0[:, fin:2 * fin].T
    wr = w0[:, 2 * fin]
    if ed == 1:
        u = w0[:, 2 * fin + 1]
        cvec = p["e0"]["b"]
    else:
        we = w0[:, 2 * fin + 1:]
        u = we @ eemb_w[:, 0]
        cvec = p["e0"]["b"] + we @ eemb_b
    aux = jnp.zeros((8, H_), jnp.float32)
    aux = aux.at[0].set(wr).at[1].set(u).at[2].set(cvec)
    aux = aux.at[3].set(p["e1"]["b"]).at[4].set(p["c0"]["b"])
    n0w = p["n0"]["w"]
    return dict(
        wa=wa, wb=wb, aux=aux,
        w1=p["e1"]["w"].T, wc0=p["c0"]["w"].T, c2=p["c2w"].T,
        w0a=n0w[:, :fin].T, w0b=n0w[:, fin:].T, b0=p["n0"]["b"][None],
        w1n=p["n1"]["w"].T, b1n=p["n1"]["b"][None],
    )


def _egcl_layer(h, coord, row, col, ea2, zer, prep, film, normalize):
    (ta, tb), hp = _k2_node_pre(h, coord, prep["wa"], prep["wb"], film)
    gg = _sc_gather_fused(ta, tb, row, col)
    m = _k3_edge(gg, ea2, prep["aux"], prep["w1"], prep["wc0"], prep["c2"],
                 normalize)
    parts = _sc_scatter(m, row, zer)
    return _k4_node_post(hp, coord, parts[0], parts[1], prep["w0a"],
                         prep["w0b"], prep["b0"], prep["w1n"], prep["b1n"])


def kernel(x, edge_index, edge_attr, x_coord, cond, timesteps, batch, params):
    xf = x.reshape(N_, NF_ * PH_)
    cf = cond[:, :, :-1].reshape(N_, OH_ * CF_)
    ids2 = cond[:, 0, -1].astype(jnp.int32)[:, None]
    batch2 = batch.astype(jnp.int32)[:, None]
    ts2 = timesteps.astype(jnp.int32)[:, None]
    sl = jnp.arange(N_, dtype=jnp.int32)
    pad_e = EPAD - EE_
    rowr = jnp.concatenate([edge_index[0].astype(jnp.int32), sl])
    colr = jnp.concatenate([edge_index[1].astype(jnp.int32), sl])
    ear = jnp.concatenate([edge_attr, jnp.zeros((N_,), jnp.float32)])
    order = jnp.argsort(rowr)
    row = jnp.concatenate([rowr[order], jnp.zeros((pad_e,), jnp.int32)])
    col = jnp.concatenate([colr[order], jnp.zeros((pad_e,), jnp.int32)])
    ea2 = jnp.concatenate([ear[order], jnp.zeros((pad_e,), jnp.float32)])[:, None]
    zer = jnp.zeros((N_, D_), jnp.float32)
    p = params

    preps_c = [_egcl_prep(p["c_gcl"][l], H_, 1, None, None) for l in range(CNL_)]
    preps_m = [_egcl_prep(p["gcl"][l], F_, H_, p["edge_emb"]["w"],
                          p["edge_emb"]["b"]) for l in range(NL_)]

    te_b = _k0_time_embed(ts2, _pe_table(), p["d0"]["w"].T, p["d0"]["b"][None],
                          p["d1"]["w"].T, p["d1"]["b"][None])
    hv, hc = _k1_prelude(xf, cf, ids2, batch2, te_b,
                         p["node_emb"]["w"].T, p["node_emb"]["b"][None],
                         p["c_in"]["w"][:, :OH_ * CF_].T,
                         p["id_emb"] @ p["c_in"]["w"][:, OH_ * CF_:].T,
                         p["c_in"]["b"][None])

    xc = x_coord
    for l in range(CNL_):
        hc, xc = _egcl_layer(hc, xc, row, col, ea2, zer, preps_c[l],
                             None, False)

    g = _k5a_pool(hc, batch2, p["c_out"]["w"].T, p["c_out"]["b"][None])
    embed = _k5b_film_tables(g, p["c_fc"]["w"].T, p["c_fc"]["b"][None])
    embed = embed.reshape(NL_, B_, 2, F_)
    scales = embed[:, :, 0, :]
    biases = embed[:, :, 1, :]

    xv = x_coord
    for l in range(NL_):
        hv, xv = _egcl_layer(hv, xv, row, col, ea2, zer, preps_m[l],
                             (scales[l], biases[l], batch2), True)

    o = _k6_out(hv, p["p0"]["w"].T, p["p0"]["b"][None],
                p["p1"]["w"].T, p["p1"]["b"][None],
                p["p2"]["w"].T, p["p2"]["b"][None])
    return o.reshape(N_, PH_, NF_), xv


# speedup vs baseline: 1.0585x; 1.0585x over previous
import functools

import jax
import jax.numpy as jnp
import numpy as np
import math
from jax import lax
from jax.experimental import pallas as pl
from jax.experimental.pallas import tpu as pltpu
from jax.experimental.pallas import tpu_sc as plsc

N_ = 10000
E_ = 160000
B_ = 16
NF_, PH_, OH_, CF_ = 3, 16, 2, 8
H_, DE_, NL_, CNL_ = 128, 32, 5, 3
EMB_, MAXE_, T_ = 16, 30, 200
F_ = H_ + DE_
EE_ = E_ + N_
D_ = 144

NTILES = 32
CH_G = 112
KCH_G = 48
EPAD = NTILES * CH_G * KCH_G
CH_S = 96
KCH_S = EPAD // 2 // 16 // CH_S
RP_ = N_ // 16

NBLK = 2000
EBLK = 512


def _mesh():
    return plsc.VectorSubcoreMesh(core_axis_name="c", subcore_axis_name="s")


def _sc_gather_fused(ta, tb, row, col):
    NV = D_ // 16

    @functools.partial(
        pl.kernel,
        mesh=_mesh(),
        compiler_params=pltpu.CompilerParams(use_tc_tiling_on_sc=False),
        out_type=jax.ShapeDtypeStruct((EPAD, D_), jnp.float32),
        scratch_types=(
            [pltpu.VMEM((CH_G,), jnp.int32)] * 8
            + [pltpu.VMEM((CH_G, D_), jnp.float32)] * 6
            + [pltpu.SemaphoreType.DMA] * 14
        ),
    )
    def k(ta_h, tb_h, row_h, col_h, g_h, *scr):
        ira = scr[0:4]
        irb = scr[4:8]
        bufa = scr[8:12]
        bufb = scr[12:14]
        isem = scr[14:18]
        ga = scr[18:22]
        gb = scr[22:24]
        wsem = scr[24:28]
        wid = lax.axis_index("s") * 2 + lax.axis_index("c")
        base_w = wid * (CH_G * KCH_G)

        def cbase(c):
            return base_w + c * CH_G

        def vadd(ba, bb):
            def vrow(r, carry):
                for cc in range(NV):
                    sl = pl.ds(cc * 16, 16)
                    ba[r, sl] = ba[r, sl] + bb[r, sl]
                return carry
            lax.fori_loop(0, CH_G, vrow, 0, unroll=2)

        def start_idx(c, u):
            pltpu.async_copy(row_h.at[pl.ds(cbase(c), CH_G)], ira[u], isem[u])
            pltpu.async_copy(col_h.at[pl.ds(cbase(c), CH_G)], irb[u], isem[u])

        def wait_idx(c, u):
            pltpu.make_async_copy(row_h.at[pl.ds(cbase(c), CH_G)], ira[u], isem[u]).wait()
            pltpu.make_async_copy(col_h.at[pl.ds(cbase(c), CH_G)], irb[u], isem[u]).wait()

        def start_gather(u2, u):
            pltpu.async_copy(ta_h.at[ira[u]], bufa[u], ga[u])
            pltpu.async_copy(tb_h.at[irb[u]], bufb[u2], gb[u2])

        def wait_gather(u2, u):
            pltpu.make_async_copy(ta_h.at[ira[u]], bufa[u], ga[u]).wait()
            pltpu.make_async_copy(tb_h.at[irb[u]], bufb[u2], gb[u2]).wait()

        def start_write(c, u):
            pltpu.async_copy(bufa[u], g_h.at[pl.ds(cbase(c), CH_G)], wsem[u])

        def wait_write(u):
            pltpu.make_async_copy(bufa[u], g_h.at[pl.ds(0, CH_G)], wsem[u]).wait()

        for u in range(2):
            pltpu.sync_copy(row_h.at[pl.ds(cbase(u), CH_G)], ira[u])
            pltpu.sync_copy(col_h.at[pl.ds(cbase(u), CH_G)], irb[u])
        start_idx(2, 2)
        start_idx(3, 3)
        start_gather(0, 0)
        start_gather(1, 1)
        scrib = EE_ + wid * (2 * CH_G)
        pltpu.async_copy(bufa[2], g_h.at[pl.ds(scrib, CH_G)], wsem[2])
        pltpu.async_copy(bufa[3], g_h.at[pl.ds(scrib + CH_G, CH_G)], wsem[3])

        def ops(c, u, do_bg):
            wait_gather(u % 2, u)
            if do_bg:
                start_idx(c + 4, u)
            vadd(bufa[u], bufb[u % 2])
            start_write(c, u)
            if do_bg:
                wait_idx(c + 2, (u + 2) % 4)
                wait_write((u + 2) % 4)
                start_gather(u % 2, (u + 2) % 4)

        def body(p, carry):
            for u in range(4):
                ops(p * 4 + u, u, True)
            return carry

        lax.fori_loop(0, (KCH_G - 4) // 4, body, 0)

        for u in range(4):
            c = KCH_G - 4 + u
            wait_gather(u % 2, u)
            vadd(bufa[u], bufb[u % 2])
            if u < 2:
                wait_idx(c + 2, (u + 2) % 4)
                wait_write((u + 2) % 4)
                start_gather(u % 2, (u + 2) % 4)
            start_write(c, u)
        for u in range(4):
            wait_write(u)

    return k(ta, tb, row, col)


def _sc_scatter(vals, row, zer):
    @functools.partial(
        pl.kernel,
        mesh=_mesh(),
        compiler_params=pltpu.CompilerParams(use_tc_tiling_on_sc=False),
        out_type=jax.ShapeDtypeStruct((2, N_, D_), jnp.float32),
        scratch_types=[
            pltpu.VMEM((CH_S,), jnp.int32),
            pltpu.VMEM((CH_S,), jnp.int32),
            pltpu.VMEM((2 * CH_S, D_), jnp.float32),
            pltpu.VMEM_SHARED((N_, D_), jnp.float32),
            pltpu.SemaphoreType.DMA,
            pltpu.SemaphoreType.DMA,
        ],
    )
    def k(vals_h, row_h, zer_h, out_h, idx0, idx1, vbuf, acc, ls0, ls1):
        cid = lax.axis_index("c")
        sid = lax.axis_index("s")
        pltpu.sync_copy(zer_h.at[pl.ds(sid * RP_, RP_)], acc.at[pl.ds(sid * RP_, RP_)])
        plsc.subcore_barrier()
        base_c = cid * (EPAD // 2) + sid * (CH_S * KCH_S)
        idx = (idx0, idx1)
        sem = (ls0, ls1)

        def cbase(c):
            return base_c + c * CH_S

        def start_loads(c, s):
            pltpu.async_copy(row_h.at[pl.ds(cbase(c), CH_S)], idx[s], sem[s])
            pltpu.async_copy(vals_h.at[pl.ds(cbase(c), CH_S)],
                             vbuf.at[pl.ds(s * CH_S, CH_S)], sem[s])

        def wait_loads(s):
            pltpu.make_async_copy(row_h.at[pl.ds(0, CH_S)], idx[s], sem[s]).wait()
            pltpu.make_async_copy(vals_h.at[pl.ds(0, CH_S)],
                                  vbuf.at[pl.ds(s * CH_S, CH_S)], sem[s]).wait()

        def scat(s):
            pltpu.sync_copy(vbuf.at[pl.ds(s * CH_S, CH_S)], acc.at[idx[s]], add=True)

        start_loads(0, 0)

        def body(p, carry):
            start_loads(2 * p + 1, 1)
            wait_loads(0)
            scat(0)
            start_loads(2 * p + 2, 0)
            wait_loads(1)
            scat(1)
            return carry

        lax.fori_loop(0, KCH_S // 2 - 1, body, 0)
        start_loads(KCH_S - 1, 1)
        wait_loads(0)
        scat(0)
        wait_loads(1)
        scat(1)
        plsc.subcore_barrier()

        def wb(j, carry):
            rb = pl.multiple_of(sid * RP_ + j * 125, 125)
            pltpu.sync_copy(acc.at[pl.ds(rb, 125)], vbuf.at[pl.ds(0, 125)])
            pltpu.sync_copy(vbuf.at[pl.ds(0, 125)], out_h.at[cid, pl.ds(rb, 125)])
            return carry

        lax.fori_loop(0, RP_ // 125, wb, 0)

    return k(vals, row, zer)


def _full(shape):
    return pl.BlockSpec(shape, lambda *_: tuple(0 for _ in shape))


def _rows(blk, width):
    return pl.BlockSpec((blk, width), lambda i: (i, 0))


def _silu(x):
    return x * jax.nn.sigmoid(x)


def _k0_time_embed(ts2, pe, d0w, d0b, d1w, d1b):
    def body(ts_r, pe_r, d0w_r, d0b_r, d1w_r, d1b_r, o_r):
        oh = (ts_r[...] == lax.broadcasted_iota(jnp.int32, (B_, T_), 1))
        pet = jnp.where(oh, 1.0, 0.0) @ pe_r[...]

        def mish(v):
            return v * jnp.tanh(jax.nn.softplus(v))

        z = mish(pet @ d0w_r[...] + d0b_r[...])
        o_r[...] = z @ d1w_r[...] + d1b_r[...]

    return pl.pallas_call(
        body,
        out_shape=jax.ShapeDtypeStruct((B_, DE_), jnp.float32),
        in_specs=[_full((B_, 1)), _full((T_, DE_)), _full((DE_, DE_ * 4)),
                  _full((1, DE_ * 4)), _full((DE_ * 4, DE_)), _full((1, DE_))],
        out_specs=_full((B_, DE_)),
    )(ts2, pe, d0w, d0b, d1w, d1b)


def _k1_prelude(xf, cf, ids2, batch2, te_b, wn, bn, wc1, wc2f, bc):
    def body(xf_r, cf_r, ids_r, b_r, te_r, wn_r, bn_r, wc1_r, wc2f_r, bc_r, hv_r, hc_r):
        a = xf_r[...] @ wn_r[...] + bn_r[...]
        ohb = jnp.where(b_r[...] == lax.broadcasted_iota(jnp.int32, (NBLK, B_), 1), 1.0, 0.0)
        te = ohb @ te_r[...]
        hv_r[...] = jnp.concatenate([a, te], axis=1)
        ohi = jnp.where(ids_r[...] == lax.broadcasted_iota(jnp.int32, (NBLK, MAXE_), 1), 1.0, 0.0)
        hc_r[...] = cf_r[...] @ wc1_r[...] + ohi @ wc2f_r[...] + bc_r[...]

    return pl.pallas_call(
        body,
        grid=(N_ // NBLK,),
        out_shape=[jax.ShapeDtypeStruct((N_, F_), jnp.float32),
                   jax.ShapeDtypeStruct((N_, H_), jnp.float32)],
        in_specs=[_rows(NBLK, NF_ * PH_), _rows(NBLK, OH_ * CF_), _rows(NBLK, 1),
                  _rows(NBLK, 1), _full((B_, DE_)), _full((NF_ * PH_, H_)),
                  _full((1, H_)), _full((OH_ * CF_, H_)), _full((MAXE_, H_)),
                  _full((1, H_))],
        out_specs=[_rows(NBLK, F_), _rows(NBLK, H_)],
    )(xf, cf, ids2, batch2, te_b, wn, bn, wc1, wc2f, bc)


def _k2_node_pre(h, coord, wa, wb, film):
    fin = h.shape[1]

    if film is None:
        def body(h_r, c_r, wa_r, wb_r, ta_r, tb_r):
            hp = h_r[...]
            cpad = jnp.pad(c_r[...], ((0, 0), (0, D_ - H_ - 3)))
            ta_r[...] = jnp.concatenate([hp @ wa_r[...], cpad], axis=1)
            tb_r[...] = jnp.concatenate([hp @ wb_r[...], -cpad], axis=1)

        return pl.pallas_call(
            body,
            grid=(N_ // NBLK,),
            out_shape=[jax.ShapeDtypeStruct((N_, D_), jnp.float32),
                       jax.ShapeDtypeStruct((N_, D_), jnp.float32)],
            in_specs=[_rows(NBLK, fin), _rows(NBLK, 3), _full((fin, H_)), _full((fin, H_))],
            out_specs=[_rows(NBLK, D_), _rows(NBLK, D_)],
        )(h, coord, wa, wb), h

    sc_l, bi_l, batch2 = film

    def body(h_r, c_r, b_r, sc_r, bi_r, wa_r, wb_r, ta_r, tb_r, hp_r):
        ohb = jnp.where(b_r[...] == lax.broadcasted_iota(jnp.int32, (NBLK, B_), 1), 1.0, 0.0)
        hp = (ohb @ sc_r[...]) * h_r[...] + ohb @ bi_r[...]
        hp_r[...] = hp
        cpad = jnp.pad(c_r[...], ((0, 0), (0, D_ - H_ - 3)))
        ta_r[...] = jnp.concatenate([hp @ wa_r[...], cpad], axis=1)
        tb_r[...] = jnp.concatenate([hp @ wb_r[...], -cpad], axis=1)

    ta, tb, hp = pl.pallas_call(
        body,
        grid=(N_ // NBLK,),
        out_shape=[jax.ShapeDtypeStruct((N_, D_), jnp.float32),
                   jax.ShapeDtypeStruct((N_, D_), jnp.float32),
                   jax.ShapeDtypeStruct((N_, fin), jnp.float32)],
        in_specs=[_rows(NBLK, fin), _rows(NBLK, 3), _rows(NBLK, 1),
                  _full((B_, fin)), _full((B_, fin)), _full((fin, H_)), _full((fin, H_))],
        out_specs=[_rows(NBLK, D_), _rows(NBLK, D_), _rows(NBLK, fin)],
    )(h, coord, batch2, sc_l, bi_l, wa, wb)
    return (ta, tb), hp


def _k3_edge(gg, ea2, aux, w1, wc0, c2, normalize):
    def body(g_r, ea_r, aux_r, w1_r, wc0_r, c2_r, m_r):
        pid = pl.program_id(0)
        g = g_r[...]
        cd = g[:, H_:H_ + 3]
        radial = jnp.sum(cd * cd, axis=1, keepdims=True)
        aux = aux_r[...]
        g0 = (g[:, :H_] + radial * aux[0:1, :] + ea_r[...] * aux[1:2, :]
              + aux[2:3, :])
        m0 = _silu(g0)
        m = _silu(m0 @ w1_r[...] + aux[3:4, :])
        q = _silu(m @ wc0_r[...] + aux[4:5, :])
        t = q @ c2_r[...]
        if normalize:
            cdn = cd / (jnp.sqrt(radial) + 1e-8)
        else:
            cdn = cd
        trans = cdn * t
        gi = pid * EBLK + lax.broadcasted_iota(jnp.int32, (EBLK, 1), 0)
        valid = jnp.where(gi < EE_, 1.0, 0.0)
        tail = jnp.pad(jnp.concatenate([trans, valid], axis=1),
                       ((0, 0), (0, D_ - H_ - 4)))
        full = jnp.concatenate([m, tail], axis=1)
        m_r[...] = jnp.where(gi < EE_, full, 0.0)

    return pl.pallas_call(
        body,
        grid=(EPAD // EBLK,),
        out_shape=jax.ShapeDtypeStruct((EPAD, D_), jnp.float32),
        in_specs=[_rows(EBLK, D_), _rows(EBLK, 1),
                  _full((8, H_)), _full((H_, H_)), _full((H_, H_)), _full((H_, 1))],
        out_specs=_rows(EBLK, D_),
    )(gg, ea2, aux, w1, wc0, c2)


def _k4_node_post(h, coord, p0, p1, w0a, w0b, b0, w1n, b1n):
    fin = h.shape[1]

    def body(h_r, c_r, p0_r, p1_r, w0a_r, w0b_r, b0_r, w1_r, b1_r, ho_r, co_r):
        agg = p0_r[...] + p1_r[...]
        cnt = jnp.maximum(agg[:, H_ + 3:H_ + 4], 1.0)
        co_r[...] = c_r[...] + agg[:, H_:H_ + 3] / cnt
        z = _silu(h_r[...] @ w0a_r[...] + agg[:, :H_] @ w0b_r[...] + b0_r[...])
        ho_r[...] = h_r[...] + z @ w1_r[...] + b1_r[...]

    return pl.pallas_call(
        body,
        grid=(N_ // NBLK,),
        out_shape=[jax.ShapeDtypeStruct((N_, fin), jnp.float32),
                   jax.ShapeDtypeStruct((N_, 3), jnp.float32)],
        in_specs=[_rows(NBLK, fin), _rows(NBLK, 3), _rows(NBLK, D_), _rows(NBLK, D_),
                  _full((fin, H_)), _full((H_, H_)), _full((1, H_)),
                  _full((H_, fin)), _full((1, fin))],
        out_specs=[_rows(NBLK, fin), _rows(NBLK, 3)],
    )(h, coord, p0, p1, w0a, w0b, b0, w1n, b1n)


def _k42_post_pre(h, coord, p0, p1, w0a, w0b, b0, w1n, b1n, wa, wb, film):
    fin = h.shape[1]

    if film is None:
        def body(h_r, c_r, p0_r, p1_r, w0a_r, w0b_r, b0_r, w1_r, b1_r,
                 wa_r, wb_r, ho_r, co_r, ta_r, tb_r):
            agg = p0_r[...] + p1_r[...]
            cnt = jnp.maximum(agg[:, H_ + 3:H_ + 4], 1.0)
            co = c_r[...] + agg[:, H_:H_ + 3] / cnt
            co_r[...] = co
            z = _silu(h_r[...] @ w0a_r[...] + agg[:, :H_] @ w0b_r[...] + b0_r[...])
            hn = h_r[...] + z @ w1_r[...] + b1_r[...]
            ho_r[...] = hn
            cpad = jnp.pad(co, ((0, 0), (0, D_ - H_ - 3)))
            ta_r[...] = jnp.concatenate([hn @ wa_r[...], cpad], axis=1)
            tb_r[...] = jnp.concatenate([hn @ wb_r[...], -cpad], axis=1)

        return pl.pallas_call(
            body,
            grid=(N_ // NBLK,),
            out_shape=[jax.ShapeDtypeStruct((N_, fin), jnp.float32),
                       jax.ShapeDtypeStruct((N_, 3), jnp.float32),
                       jax.ShapeDtypeStruct((N_, D_), jnp.float32),
                       jax.ShapeDtypeStruct((N_, D_), jnp.float32)],
            in_specs=[_rows(NBLK, fin), _rows(NBLK, 3), _rows(NBLK, D_),
                      _rows(NBLK, D_), _full((fin, H_)), _full((H_, H_)),
                      _full((1, H_)), _full((H_, fin)), _full((1, fin)),
                      _full((fin, H_)), _full((fin, H_))],
            out_specs=[_rows(NBLK, fin), _rows(NBLK, 3),
                       _rows(NBLK, D_), _rows(NBLK, D_)],
        )(h, coord, p0, p1, w0a, w0b, b0, w1n, b1n, wa, wb)

    sc_l, bi_l, batch2 = film

    def body(h_r, c_r, p0_r, p1_r, b_r, sc_r, bi_r, w0a_r, w0b_r, b0_r,
             w1_r, b1_r, wa_r, wb_r, ho_r, co_r, ta_r, tb_r):
        agg = p0_r[...] + p1_r[...]
        cnt = jnp.maximum(agg[:, H_ + 3:H_ + 4], 1.0)
        co = c_r[...] + agg[:, H_:H_ + 3] / cnt
        co_r[...] = co
        z = _silu(h_r[...] @ w0a_r[...] + agg[:, :H_] @ w0b_r[...] + b0_r[...])
        hn = h_r[...] + z @ w1_r[...] + b1_r[...]
        ohb = jnp.where(b_r[...] == lax.broadcasted_iota(jnp.int32, (NBLK, B_), 1), 1.0, 0.0)
        hp = (ohb @ sc_r[...]) * hn + ohb @ bi_r[...]
        ho_r[...] = hp
        cpad = jnp.pad(co, ((0, 0), (0, D_ - H_ - 3)))
        ta_r[...] = jnp.concatenate([hp @ wa_r[...], cpad], axis=1)
        tb_r[...] = jnp.concatenate([hp @ wb_r[...], -cpad], axis=1)

    return pl.pallas_call(
        body,
        grid=(N_ // NBLK,),
        out_shape=[jax.ShapeDtypeStruct((N_, fin), jnp.float32),
                   jax.ShapeDtypeStruct((N_, 3), jnp.float32),
                   jax.ShapeDtypeStruct((N_, D_), jnp.float32),
                   jax.ShapeDtypeStruct((N_, D_), jnp.float32)],
        in_specs=[_rows(NBLK, fin), _rows(NBLK, 3), _rows(NBLK, D_),
                  _rows(NBLK, D_), _rows(NBLK, 1), _full((B_, fin)),
                  _full((B_, fin)), _full((fin, H_)), _full((H_, H_)),
                  _full((1, H_)), _full((H_, fin)), _full((1, fin)),
                  _full((fin, H_)), _full((fin, H_))],
        out_specs=[_rows(NBLK, fin), _rows(NBLK, 3),
                   _rows(NBLK, D_), _rows(NBLK, D_)],
    )(h, coord, p0, p1, batch2, sc_l, bi_l, w0a, w0b, b0, w1n, b1n, wa, wb)


def _k5a_pool(hc, batch2, wco, bco):
    def body(hc_r, b_r, wco_r, bco_r, g_r):
        i = pl.program_id(0)
        hco = hc_r[...] @ wco_r[...] + bco_r[...]
        bv = b_r[...]
        parts = []
        for b in range(B_):
            mb = jnp.where(bv == b, hco, -jnp.inf)
            parts.append(jnp.max(mb, axis=0, keepdims=True))
        cur = jnp.concatenate(parts, axis=0)

        @pl.when(i == 0)
        def _():
            g_r[...] = jnp.full((B_, H_), -jnp.inf, jnp.float32)

        g_r[...] = jnp.maximum(g_r[...], cur)

    return pl.pallas_call(
        body,
        grid=(N_ // NBLK,),
        out_shape=jax.ShapeDtypeStruct((B_, H_), jnp.float32),
        in_specs=[_rows(NBLK, H_), _rows(NBLK, 1), _full((H_, H_)), _full((1, H_))],
        out_specs=_full((B_, H_)),
    )(hc, batch2, wco, bco)


def _k5b_film_tables(g, wfc, bfc):
    cc = 2 * NL_ * F_

    def body(g_r, w_r, b_r, o_r):
        gv = g_r[...]
        gv = jnp.where(gv > -1e30, gv, 0.0)
        o_r[...] = gv @ w_r[...] + b_r[...]

    return pl.pallas_call(
        body,
        out_shape=jax.ShapeDtypeStruct((B_, cc), jnp.float32),
        in_specs=[_full((B_, H_)), _full((H_, cc)), _full((1, cc))],
        out_specs=_full((B_, cc)),
    )(g, wfc, bfc)


def _k6_out(h, w0, b0, w1, b1, w2, b2):
    def body(h_r, w0_r, b0_r, w1_r, b1_r, w2_r, b2_r, o_r):
        z = jnp.maximum(h_r[...] @ w0_r[...] + b0_r[...], 0.0)
        z = jnp.maximum(z @ w1_r[...] + b1_r[...], 0.0)
        o_r[...] = z @ w2_r[...] + b2_r[...]

    return pl.pallas_call(
        body,
        grid=(N_ // NBLK,),
        out_shape=jax.ShapeDtypeStruct((N_, NF_ * PH_), jnp.float32),
        in_specs=[_rows(NBLK, F_), _full((F_, H_)), _full((1, H_)),
                  _full((H_, H_)), _full((1, H_)), _full((H_, NF_ * PH_)),
                  _full((1, NF_ * PH_))],
        out_specs=_rows(NBLK, NF_ * PH_),
    )(h, w0, b0, w1, b1, w2, b2)


def _pe_table():
    pes = np.zeros((T_, DE_), dtype=np.float32)
    pos = np.arange(T_, dtype=np.float32)[:, None] + 1.0
    div = np.exp(np.arange(0, DE_, 2, dtype=np.float32) * (-math.log(10000.0) / DE_))
    pes[:, 0::2] = np.sin(pos * div)
    pes[:, 1::2] = np.cos(pos * div)
    return jnp.asarray(pes)


def _egcl_prep(p, fin, ed, eemb_w, eemb_b):
    w0 = p["e0"]["w"]
    wa = w0[:, :fin].T
    wb = w0[:, fin:2 * fin].T
    wr = w0[:, 2 * fin]
    if ed == 1:
        u = w0[:, 2 * fin + 1]
        cvec = p["e0"]["b"]
    else:
        we = w0[:, 2 * fin + 1:]
        u = we @ eemb_w[:, 0]
        cvec = p["e0"]["b"] + we @ eemb_b
    aux = jnp.zeros((8, H_), jnp.float32)
    aux = aux.at[0].set(wr).at[1].set(u).at[2].set(cvec)
    aux = aux.at[3].set(p["e1"]["b"]).at[4].set(p["c0"]["b"])
    n0w = p["n0"]["w"]
    return dict(
        wa=wa, wb=wb, aux=aux,
        w1=p["e1"]["w"].T, wc0=p["c0"]["w"].T, c2=p["c2w"].T,
        w0a=n0w[:, :fin].T, w0b=n0w[:, fin:].T, b0=p["n0"]["b"][None],
        w1n=p["n1"]["w"].T, b1n=p["n1"]["b"][None],
    )


def _egnn_chain(h, coord, row, col, ea2, zer, preps, films, normalize):
    nl = len(preps)
    (ta, tb), h = _k2_node_pre(h, coord, preps[0]["wa"], preps[0]["wb"], films[0])
    for l in range(nl):
        p = preps[l]
        gg = _sc_gather_fused(ta, tb, row, col)
        m = _k3_edge(gg, ea2, p["aux"], p["w1"], p["wc0"], p["c2"], normalize)
        parts = _sc_scatter(m, row, zer)
        if l + 1 < nl:
            pn = preps[l + 1]
            h, coord, ta, tb = _k42_post_pre(
                h, coord, parts[0], parts[1], p["w0a"], p["w0b"], p["b0"],
                p["w1n"], p["b1n"], pn["wa"], pn["wb"], films[l + 1])
        else:
            h, coord = _k4_node_post(h, coord, parts[0], parts[1], p["w0a"],
                                     p["w0b"], p["b0"], p["w1n"], p["b1n"])
    return h, coord


def kernel(x, edge_index, edge_attr, x_coord, cond, timesteps, batch, params):
    xf = x.reshape(N_, NF_ * PH_)
    cf = cond[:, :, :-1].reshape(N_, OH_ * CF_)
    ids2 = cond[:, 0, -1].astype(jnp.int32)[:, None]
    batch2 = batch.astype(jnp.int32)[:, None]
    ts2 = timesteps.astype(jnp.int32)[:, None]
    sl = jnp.arange(N_, dtype=jnp.int32)
    pad_e = EPAD - EE_
    row = jnp.concatenate([edge_index[0].astype(jnp.int32), sl,
                           jnp.zeros((pad_e,), jnp.int32)])
    col = jnp.concatenate([edge_index[1].astype(jnp.int32), sl,
                           jnp.zeros((pad_e,), jnp.int32)])
    ea2 = jnp.concatenate([edge_attr, jnp.zeros((N_ + pad_e,), jnp.float32)])[:, None]
    zer = jnp.zeros((N_, D_), jnp.float32)
    p = params

    preps_c = [_egcl_prep(p["c_gcl"][l], H_, 1, None, None) for l in range(CNL_)]
    preps_m = [_egcl_prep(p["gcl"][l], F_, H_, p["edge_emb"]["w"],
                          p["edge_emb"]["b"]) for l in range(NL_)]

    te_b = _k0_time_embed(ts2, _pe_table(), p["d0"]["w"].T, p["d0"]["b"][None],
                          p["d1"]["w"].T, p["d1"]["b"][None])
    hv, hc = _k1_prelude(xf, cf, ids2, batch2, te_b,
                         p["node_emb"]["w"].T, p["node_emb"]["b"][None],
                         p["c_in"]["w"][:, :OH_ * CF_].T,
                         p["id_emb"] @ p["c_in"]["w"][:, OH_ * CF_:].T,
                         p["c_in"]["b"][None])

    hc, xc = _egnn_chain(hc, x_coord, row, col, ea2, zer, preps_c,
                         [None] * CNL_, False)

    g = _k5a_pool(hc, batch2, p["c_out"]["w"].T, p["c_out"]["b"][None])
    embed = _k5b_film_tables(g, p["c_fc"]["w"].T, p["c_fc"]["b"][None])
    embed = embed.reshape(NL_, B_, 2, F_)
    scales = embed[:, :, 0, :]
    biases = embed[:, :, 1, :]

    hv, xv = _egnn_chain(hv, x_coord, row, col, ea2, zer, preps_m,
                         [(scales[l], biases[l], batch2) for l in range(NL_)],
                         True)

    o = _k6_out(hv, p["p0"]["w"].T, p["p0"]["b"][None],
                p["p1"]["w"].T, p["p1"]["b"][None],
                p["p2"]["w"].T, p["p2"]["b"][None])
    return o.reshape(N_, PH_, NF_), xv
